# baseline (speedup 1.0000x reference)
import functools

import jax
import jax.numpy as jnp
from jax import lax
from jax.experimental import pallas as pl
from jax.experimental.pallas import tpu as pltpu
from jax.experimental.pallas import tpu_sc as plsc

B = 16384
NC = 26
V = 100000
D = 16
CONT = 13
EPS = 1e-5

NW = 32
BPW = B // NW
CH = 128
NCHUNK = BPW // CH
ROWS = CH * NC

BT = 512
T = B // BT


def _sc_gather(W_emb, W_lin_flat, idx_flat, idx_t_flat):
    mesh = plsc.VectorSubcoreMesh(core_axis_name="c", subcore_axis_name="s")

    @functools.partial(
        pl.kernel,
        mesh=mesh,
        compiler_params=pltpu.CompilerParams(use_tc_tiling_on_sc=False),
        out_type=(
            jax.ShapeDtypeStruct((B * NC, D), jnp.float32),
            jax.ShapeDtypeStruct((B,), jnp.float32),
        ),
        scratch_types=[
            pltpu.VMEM((ROWS,), jnp.int32),
            pltpu.VMEM((ROWS, D), jnp.float32),
            pltpu.VMEM((NC * BPW,), jnp.int32),
            pltpu.VMEM((NC * BPW,), jnp.float32),
            pltpu.VMEM((BPW,), jnp.float32),
            pltpu.SemaphoreType.DMA,
            pltpu.SemaphoreType.DMA,
        ],
    )
    def k(emb_hbm, lin_hbm, idx_hbm, idxt_hbm, emb_out, lin_out,
          idx_v, rows_v, idx2_v, lin_v, acc_v, sem, sem2):
        wid = lax.axis_index("s") * 2 + lax.axis_index("c")
        base = wid * BPW
        for c in range(NC):
            pltpu.sync_copy(idxt_hbm.at[pl.ds(c * B + base, BPW)],
                            idx2_v.at[pl.ds(c * BPW, BPW)])
        lin_cp = pltpu.async_copy(lin_hbm.at[idx2_v], lin_v, sem2)
        for kk in range(NCHUNK):
            o = base * NC + kk * ROWS
            pltpu.sync_copy(idx_hbm.at[pl.ds(o, ROWS)], idx_v)
            pltpu.async_copy(emb_hbm.at[idx_v], rows_v, sem).wait()
            pltpu.sync_copy(rows_v, emb_out.at[pl.ds(o, ROWS)])
        lin_cp.wait()
        for j in range(BPW // 16):
            a = lin_v[pl.ds(j * 16, 16)]
            for c in range(1, NC):
                a = a + lin_v[pl.ds(c * BPW + j * 16, 16)]
            acc_v[pl.ds(j * 16, 16)] = a
        pltpu.sync_copy(acc_v, lin_out.at[pl.ds(base, BPW)])

    return k(W_emb, W_lin_flat, idx_flat, idx_t_flat)


def _stage1_body(xc, emb, lin, w1c, w1e, b1r, wcr, wfmt, b4s,
                 a1_ref, fm_ref, st_ref):
    pid = pl.program_id(0)
    x = xc[...]
    e = emb[...]
    a1 = (jnp.dot(x, w1c[...], preferred_element_type=jnp.float32)
          + jnp.dot(e, w1e[...], preferred_element_type=jnp.float32)
          + b1r[...])
    a1_ref[...] = a1

    @pl.when(pid == 0)
    def _():
        st_ref[...] = jnp.zeros_like(st_ref)

    st_ref[0:1, :] += jnp.sum(a1, axis=0, keepdims=True)
    st_ref[1:2, :] += jnp.sum(a1 * a1, axis=0, keepdims=True)

    cont_fm = jnp.dot(x, wfmt[...], preferred_element_type=jnp.float32)
    s = cont_fm
    ss = cont_fm * cont_fm
    for c in range(NC):
        ec = e[:, c * D:(c + 1) * D]
        s = s + ec
        ss = ss + ec * ec
    fm2 = 0.5 * jnp.sum(s * s - ss, axis=1, keepdims=True)
    fm1 = jnp.sum(x * wcr[...], axis=1, keepdims=True)
    fm_ref[...] = fm1 + fm2 + lin[...] + b4s[0]


def _stage_mid_body(a_in, st_in, gr, ber, wt, br, a_ref, st_ref):
    pid = pl.program_id(0)
    st = st_in[...]
    m = st[0:1, :] * (1.0 / B)
    var = st[1:2, :] * (1.0 / B) - m * m
    scale = gr[...] * lax.rsqrt(var + EPS)
    h = jnp.maximum((a_in[...] - m) * scale + ber[...], 0.0)
    a = jnp.dot(h, wt[...], preferred_element_type=jnp.float32) + br[...]
    a_ref[...] = a

    @pl.when(pid == 0)
    def _():
        st_ref[...] = jnp.zeros_like(st_ref)

    st_ref[0:1, :] += jnp.sum(a, axis=0, keepdims=True)
    st_ref[1:2, :] += jnp.sum(a * a, axis=0, keepdims=True)


def _stage4_body(a_in, st_in, gr, ber, w4r, fm_in, out_ref):
    st = st_in[...]
    m = st[0:1, :] * (1.0 / B)
    var = st[1:2, :] * (1.0 / B) - m * m
    scale = gr[...] * lax.rsqrt(var + EPS)
    h = jnp.maximum((a_in[...] - m) * scale + ber[...], 0.0)
    deep = jnp.sum(h * w4r[...], axis=1, keepdims=True)
    z = fm_in[...] + deep
    out_ref[...] = 1.0 / (1.0 + jnp.exp(-z))


def _row(i):
    return (i, 0)


def _rep(i):
    return (0, 0)


def _batch_spec(n):
    return pl.BlockSpec((BT, n), _row)


def _full_spec(shape):
    return pl.BlockSpec(shape, _rep)


def _stage1(x_cont, emb2d, lin2d, w1c, w1e, b1r, wcr, wfmt, b4):
    return pl.pallas_call(
        _stage1_body,
        grid=(T,),
        in_specs=[
            _batch_spec(CONT),
            _batch_spec(NC * D),
            _batch_spec(1),
            _full_spec((CONT, 256)),
            _full_spec((NC * D, 256)),
            _full_spec((1, 256)),
            _full_spec((1, CONT)),
            _full_spec((CONT, D)),
            pl.BlockSpec(memory_space=pltpu.SMEM),
        ],
        out_specs=[
            _batch_spec(256),
            _batch_spec(1),
            _full_spec((2, 256)),
        ],
        out_shape=[
            jax.ShapeDtypeStruct((B, 256), jnp.float32),
            jax.ShapeDtypeStruct((B, 1), jnp.float32),
            jax.ShapeDtypeStruct((2, 256), jnp.float32),
        ],
    )(x_cont, emb2d, lin2d, w1c, w1e, b1r, wcr, wfmt, b4)


def _stage_mid(a_in, st_in, gr, ber, wt, br, n_in, n_out):
    return pl.pallas_call(
        _stage_mid_body,
        grid=(T,),
        in_specs=[
            _batch_spec(n_in),
            _full_spec((2, n_in)),
            _full_spec((1, n_in)),
            _full_spec((1, n_in)),
            _full_spec((n_in, n_out)),
            _full_spec((1, n_out)),
        ],
        out_specs=[
            _batch_spec(n_out),
            _full_spec((2, n_out)),
        ],
        out_shape=[
            jax.ShapeDtypeStruct((B, n_out), jnp.float32),
            jax.ShapeDtypeStruct((2, n_out), jnp.float32),
        ],
    )(a_in, st_in, gr, ber, wt, br)


def _stage4(a_in, st_in, gr, ber, w4r, fm):
    return pl.pallas_call(
        _stage4_body,
        grid=(T,),
        in_specs=[
            _batch_spec(64),
            _full_spec((2, 64)),
            _full_spec((1, 64)),
            _full_spec((1, 64)),
            _full_spec((1, 64)),
            _batch_spec(1),
        ],
        out_specs=_batch_spec(1),
        out_shape=jax.ShapeDtypeStruct((B, 1), jnp.float32),
    )(a_in, st_in, gr, ber, w4r, fm)


def kernel(x_cont, x_cat, W_emb, W_lin_emb, Wc_lin, Wfm,
           W1, b1, g1, be1, W2, b2, g2, be2, W3, b3, g3, be3, W4, b4):
    xi = x_cat.astype(jnp.int32)
    offs = jnp.arange(NC, dtype=jnp.int32) * V
    idx_flat = (xi + offs[None, :]).reshape(-1)
    idx_t_flat = (xi.T + offs[:, None]).reshape(-1)

    emb, lin_sum = _sc_gather(W_emb, W_lin_emb.reshape(-1), idx_flat,
                              idx_t_flat)
    emb2d = emb.reshape(B, NC * D)
    lin2d = lin_sum.reshape(B, 1)

    a1, fm, st1 = _stage1(
        x_cont, emb2d, lin2d,
        W1[:, :CONT].T, W1[:, CONT:].T, b1.reshape(1, 256),
        Wc_lin, Wfm.T, b4,
    )
    a2, st2 = _stage_mid(a1, st1, g1.reshape(1, 256), be1.reshape(1, 256),
                         W2.T, b2.reshape(1, 128), 256, 128)
    a3, st3 = _stage_mid(a2, st2, g2.reshape(1, 128), be2.reshape(1, 128),
                         W3.T, b3.reshape(1, 64), 128, 64)
    out = _stage4(a3, st3, g3.reshape(1, 64), be3.reshape(1, 64),
                  W4, fm)
    return out.reshape(B)
